# baseline (speedup 1.0000x reference)
import jax
import jax.numpy as jnp
from jax.experimental import pallas as pl

_BM = 512
_NSPLIT = 4


def _split_dot(ms, rhs_ref, cast):
    n = len(ms)
    k = rhs_ref.shape[0] // n
    dims = (((1,), (0,)), ((), ()))
    acc = None
    for j, m in enumerate(ms):
        r = rhs_ref[pl.ds(j * k, k), :]
        if cast:
            r = r.astype(jnp.bfloat16)
        part = jax.lax.dot_general(m, r, dims,
                                   preferred_element_type=jnp.float32)
        acc = part if acc is None else acc + part
    return acc


def _cast_parts(refs):
    return [r[...].astype(jnp.bfloat16) for r in refs]


def _s1_body(*refs):
    uv = _cast_parts(refs[:_NSPLIT])
    i0_ref, a_ref = refs[_NSPLIT], refs[_NSPLIT + 1]
    a_ref[...] = _split_dot(uv, i0_ref, cast=True).astype(jnp.bfloat16)


def _s2_body(*refs):
    vu = _cast_parts(refs[:_NSPLIT])
    u0f_ref, u0sf_ref, ab_ref, i0_ref, bb_ref, spec_i_ref = refs[_NSPLIT:]
    b = _split_dot(vu, u0f_ref, cast=True)
    bs = _split_dot(vu, u0sf_ref, cast=True)
    d = _split_dot(vu, ab_ref, cast=False)
    f = b.shape[1]
    bb_ref[:, :f] = b.astype(jnp.bfloat16)
    bb_ref[:, f:] = bs.astype(jnp.bfloat16)
    spec_i_ref[...] = (i0_ref[...] + b + d) * (1.0 / 3.0)


def _s3_body(*refs):
    uv = _cast_parts(refs[:_NSPLIT])
    bb_ref, u0_ref, u0s_ref, ab_ref, spec_u_ref, share_u_ref = refs[_NSPLIT:]
    acc = _split_dot(uv, bb_ref, cast=False)
    f = u0_ref.shape[1]
    a = ab_ref[...].astype(jnp.float32)
    spec_u_ref[...] = (u0_ref[...] + a + acc[:, :f]) * (1.0 / 3.0)
    share_u_ref[...] = (u0s_ref[...] + a + acc[:, f:]) * (1.0 / 3.0)


def _row_spec(bm, w):
    return pl.BlockSpec((bm, w), lambda i: (i, 0))


def _half_specs(bm, k):
    def imap(j):
        return lambda i: (i, j)
    return [pl.BlockSpec((bm, k // _NSPLIT), imap(j))
            for j in range(_NSPLIT)]


def _full_spec(k, w):
    return pl.BlockSpec((k, w), lambda i: (0, 0))


def _domain(UV, VU, u0, u0_share, i0):
    n_u, f = u0.shape
    n_i = i0.shape[0]
    bm = _BM
    bf = jnp.bfloat16

    a_bf = pl.pallas_call(
        _s1_body,
        grid=(n_u // bm,),
        in_specs=_half_specs(bm, n_i) + [_full_spec(n_i, f)],
        out_specs=_row_spec(bm, f),
        out_shape=jax.ShapeDtypeStruct((n_u, f), bf),
    )(*([UV] * _NSPLIT), i0)

    bb_bf, spec_i = pl.pallas_call(
        _s2_body,
        grid=(n_i // bm,),
        in_specs=_half_specs(bm, n_u) + [
            _full_spec(n_u, f), _full_spec(n_u, f), _full_spec(n_u, f),
            _row_spec(bm, f)],
        out_specs=(_row_spec(bm, 2 * f), _row_spec(bm, f)),
        out_shape=(jax.ShapeDtypeStruct((n_i, 2 * f), bf),
                   jax.ShapeDtypeStruct((n_i, f), jnp.float32)),
    )(*([VU] * _NSPLIT), u0, u0_share, a_bf, i0)

    spec_u, share_u = pl.pallas_call(
        _s3_body,
        grid=(n_u // bm,),
        in_specs=_half_specs(bm, n_i) + [
            _full_spec(n_i, 2 * f),
            _row_spec(bm, f), _row_spec(bm, f), _row_spec(bm, f)],
        out_specs=(_row_spec(bm, f), _row_spec(bm, f)),
        out_shape=(jax.ShapeDtypeStruct((n_u, f), jnp.float32),
                   jax.ShapeDtypeStruct((n_u, f), jnp.float32)),
    )(*([UV] * _NSPLIT), bb_bf, u0, u0_share, a_bf)

    return share_u, spec_u, spec_i


def kernel(source_UV, source_VU, target_UV, target_VU, source_user_emb,
           target_user_emb, source_item_emb, target_item_emb,
           source_user_emb_share, target_user_emb_share):
    s_share_u, s_spec_u, s_spec_i = _domain(
        source_UV, source_VU, source_user_emb, source_user_emb_share,
        source_item_emb)
    t_share_u, t_spec_u, t_spec_i = _domain(
        target_UV, target_VU, target_user_emb, target_user_emb_share,
        target_item_emb)
    return (s_share_u, s_spec_u, s_spec_i, t_share_u, t_spec_u, t_spec_i)


# speedup vs baseline: 1.0326x; 1.0326x over previous
import jax
import jax.numpy as jnp
from jax.experimental import pallas as pl

_BM = 512
_NSPLIT = 2


def _split_dot(ms, rhs_ref, cast):
    n = len(ms)
    k = rhs_ref.shape[0] // n
    dims = (((1,), (0,)), ((), ()))
    acc = None
    for j, m in enumerate(ms):
        r = rhs_ref[pl.ds(j * k, k), :]
        if cast:
            r = r.astype(jnp.bfloat16)
        part = jax.lax.dot_general(m, r, dims,
                                   preferred_element_type=jnp.float32)
        acc = part if acc is None else acc + part
    return acc


def _cast_parts(refs):
    return [r[...].astype(jnp.bfloat16) for r in refs]


def _s1_one(uv_refs, i0_ref, a_ref):
    uv = _cast_parts(uv_refs)
    a_ref[...] = _split_dot(uv, i0_ref, cast=True).astype(jnp.bfloat16)


def _s1_body(*refs):
    ns = _NSPLIT
    _s1_one(refs[0:ns], refs[2 * ns], refs[2 * ns + 2])
    _s1_one(refs[ns:2 * ns], refs[2 * ns + 1], refs[2 * ns + 3])


def _s2_one(vu_refs, u0b_ref, u0sb_ref, ab_ref, i0_ref, bb_ref, spec_i_ref):
    vu = _cast_parts(vu_refs)
    b = _split_dot(vu, u0b_ref, cast=False)
    bs = _split_dot(vu, u0sb_ref, cast=False)
    d = _split_dot(vu, ab_ref, cast=False)
    f = b.shape[1]
    bb_ref[:, :f] = b.astype(jnp.bfloat16)
    bb_ref[:, f:] = bs.astype(jnp.bfloat16)
    spec_i_ref[...] = (i0_ref[...] + b + d) * (1.0 / 3.0)


def _s2_body(*refs):
    ns = _NSPLIT
    _s2_one(refs[0:ns], *refs[2 * ns:2 * ns + 4], *refs[2 * ns + 8:2 * ns + 10])
    _s2_one(refs[ns:2 * ns], *refs[2 * ns + 4:2 * ns + 8],
            *refs[2 * ns + 10:2 * ns + 12])


def _s3_one(uv_refs, bb_ref, u0_ref, u0s_ref, ab_ref, spec_u_ref,
            share_u_ref):
    uv = _cast_parts(uv_refs)
    acc = _split_dot(uv, bb_ref, cast=False)
    f = u0_ref.shape[1]
    a = ab_ref[...].astype(jnp.float32)
    u0 = u0_ref[...].astype(jnp.float32)
    u0s = u0s_ref[...].astype(jnp.float32)
    spec_u_ref[...] = (u0 + a + acc[:, :f]) * (1.0 / 3.0)
    share_u_ref[...] = (u0s + a + acc[:, f:]) * (1.0 / 3.0)


def _s3_body(*refs):
    ns = _NSPLIT
    _s3_one(refs[0:ns], *refs[2 * ns:2 * ns + 4], *refs[2 * ns + 8:2 * ns + 10])
    _s3_one(refs[ns:2 * ns], *refs[2 * ns + 4:2 * ns + 8],
            *refs[2 * ns + 10:2 * ns + 12])


def _row_spec(bm, w):
    return pl.BlockSpec((bm, w), lambda i: (i, 0))


def _split_specs(bm, k):
    def imap(j):
        return lambda i: (i, j)
    return [pl.BlockSpec((bm, k // _NSPLIT), imap(j))
            for j in range(_NSPLIT)]


def _full_spec(k, w):
    return pl.BlockSpec((k, w), lambda i: (0, 0))


def kernel(source_UV, source_VU, target_UV, target_VU, source_user_emb,
           target_user_emb, source_item_emb, target_item_emb,
           source_user_emb_share, target_user_emb_share):
    n_u, f = source_user_emb.shape
    n_i = source_item_emb.shape[0]
    bm = _BM
    ns_grid = (n_u // bm,)
    bf = jnp.bfloat16
    f32 = jnp.float32

    adj_specs = _split_specs(bm, n_i) + _split_specs(bm, n_i)

    u0b_s = source_user_emb.astype(bf)
    u0sb_s = source_user_emb_share.astype(bf)
    u0b_t = target_user_emb.astype(bf)
    u0sb_t = target_user_emb_share.astype(bf)

    a_s, a_t = pl.pallas_call(
        _s1_body,
        grid=ns_grid,
        in_specs=adj_specs + [_full_spec(n_i, f)] * 2,
        out_specs=(_row_spec(bm, f), _row_spec(bm, f)),
        out_shape=(jax.ShapeDtypeStruct((n_u, f), bf),
                   jax.ShapeDtypeStruct((n_u, f), bf)),
    )(*([source_UV] * _NSPLIT), *([target_UV] * _NSPLIT),
      source_item_emb, target_item_emb)

    bb_s, spec_i_s, bb_t, spec_i_t = pl.pallas_call(
        _s2_body,
        grid=ns_grid,
        in_specs=adj_specs + [
            _full_spec(n_u, f), _full_spec(n_u, f), _full_spec(n_u, f),
            _row_spec(bm, f),
            _full_spec(n_u, f), _full_spec(n_u, f), _full_spec(n_u, f),
            _row_spec(bm, f)],
        out_specs=(_row_spec(bm, 2 * f), _row_spec(bm, f),
                   _row_spec(bm, 2 * f), _row_spec(bm, f)),
        out_shape=(jax.ShapeDtypeStruct((n_i, 2 * f), bf),
                   jax.ShapeDtypeStruct((n_i, f), f32),
                   jax.ShapeDtypeStruct((n_i, 2 * f), bf),
                   jax.ShapeDtypeStruct((n_i, f), f32)),
    )(*([source_VU] * _NSPLIT), *([target_VU] * _NSPLIT),
      u0b_s, u0sb_s, a_s, source_item_emb,
      u0b_t, u0sb_t, a_t, target_item_emb)

    spec_u_s, share_u_s, spec_u_t, share_u_t = pl.pallas_call(
        _s3_body,
        grid=ns_grid,
        in_specs=adj_specs + [
            _full_spec(n_i, 2 * f), _row_spec(bm, f), _row_spec(bm, f),
            _row_spec(bm, f),
            _full_spec(n_i, 2 * f), _row_spec(bm, f), _row_spec(bm, f),
            _row_spec(bm, f)],
        out_specs=(_row_spec(bm, f), _row_spec(bm, f),
                   _row_spec(bm, f), _row_spec(bm, f)),
        out_shape=(jax.ShapeDtypeStruct((n_u, f), f32),
                   jax.ShapeDtypeStruct((n_u, f), f32),
                   jax.ShapeDtypeStruct((n_u, f), f32),
                   jax.ShapeDtypeStruct((n_u, f), f32)),
    )(*([source_UV] * _NSPLIT), *([target_UV] * _NSPLIT),
      bb_s, u0b_s, u0sb_s, a_s,
      bb_t, u0b_t, u0sb_t, a_t)

    return (share_u_s, spec_u_s, spec_i_s, share_u_t, spec_u_t, spec_i_t)
